# baseline (speedup 1.0000x reference)
import functools

import jax
import jax.numpy as jnp
from jax import lax
from jax.experimental import pallas as pl
from jax.experimental.pallas import tpu as pltpu

K_TOP = 64
MIN32 = -(2 ** 31)
POS_MASK = 0x7FFFFFFF


def _key_of_bits(b):
    return jnp.where(b >= 0, b, b ^ POS_MASK)


def _enc_kernel(x_ref, w_ref, keys_ref):
    enc = lax.dot_general(
        x_ref[...], w_ref[...], (((1,), (1,)), ((), ())),
        preferred_element_type=jnp.float32)
    keys_ref[...] = _key_of_bits(lax.bitcast_convert_type(enc, jnp.int32))


def _topk_mask_kernel(keys_ref, out_ref, *, bt, hidden, chunk):
    nchunks = hidden // chunk

    def count_ge(t_signed):
        def chunk_body(c, cnt):
            kc = keys_ref[:, pl.ds(c * chunk, chunk)]
            return cnt + jnp.sum((kc >= t_signed).astype(jnp.int32),
                                 axis=1, keepdims=True)
        return lax.fori_loop(0, nchunks, chunk_body,
                             jnp.zeros((bt, 1), jnp.int32))

    def bit_body(t, prefix):
        bitval = lax.shift_left(jnp.int32(1), 31 - t)
        cand = prefix | bitval
        cnt = count_ge(cand ^ MIN32)
        return jnp.where(cnt >= K_TOP, cand, prefix)

    prefix = lax.fori_loop(0, 32, bit_body, jnp.zeros((bt, 1), jnp.int32))
    t_signed = prefix ^ MIN32

    def mask_body(c, carry):
        sl = pl.ds(c * chunk, chunk)
        kc = keys_ref[:, sl]
        val = lax.bitcast_convert_type(_key_of_bits(kc), jnp.float32)
        out_ref[:, sl] = jnp.where(kc >= t_signed, val, 0.0)
        return carry

    lax.fori_loop(0, nchunks, mask_body, 0)


def _dec_kernel(s_ref, w_ref, out_ref, acc_ref, sem, *, nh, nb, bt):
    i = pl.program_id(0)
    j = pl.program_id(1)

    @pl.when((j == 0) & (i > 0))
    def _wait_prev():
        pltpu.make_async_copy(
            acc_ref, out_ref.at[pl.ds((i - 1) * bt, bt), :], sem).wait()

    prod = lax.dot_general(
        s_ref[...].astype(jnp.bfloat16), w_ref[...],
        (((1,), (1,)), ((), ())), preferred_element_type=jnp.float32)

    @pl.when(j == 0)
    def _init():
        acc_ref[...] = prod

    @pl.when(j != 0)
    def _acc():
        acc_ref[...] += prod

    @pl.when(j == nh - 1)
    def _flush():
        copy = pltpu.make_async_copy(
            acc_ref, out_ref.at[pl.ds(i * bt, bt), :], sem)
        copy.start()

        @pl.when(i == nb - 1)
        def _wait_last():
            copy.wait()


@jax.jit
def kernel(x, W_enc, W_dec):
    batch, din = x.shape
    hidden = W_enc.shape[0]

    bt = min(1024, batch)
    hb = min(512, hidden)
    keys = pl.pallas_call(
        _enc_kernel,
        grid=(batch // bt, hidden // hb),
        in_specs=[
            pl.BlockSpec((bt, din), lambda i, j: (i, 0)),
            pl.BlockSpec((hb, din), lambda i, j: (j, 0)),
        ],
        out_specs=pl.BlockSpec((bt, hb), lambda i, j: (i, j)),
        out_shape=jax.ShapeDtypeStruct((batch, hidden), jnp.int32),
        compiler_params=pltpu.CompilerParams(
            vmem_limit_bytes=63 * 1024 * 1024),
    )(x, W_enc)

    bta = min(128, batch)
    chunk = min(512, hidden)
    sparse = pl.pallas_call(
        functools.partial(_topk_mask_kernel, bt=bta, hidden=hidden,
                          chunk=chunk),
        grid=(batch // bta,),
        in_specs=[pl.BlockSpec((bta, hidden), lambda i: (i, 0))],
        out_specs=pl.BlockSpec((bta, hidden), lambda i: (i, 0)),
        out_shape=jax.ShapeDtypeStruct((batch, hidden), jnp.float32),
        compiler_params=pltpu.CompilerParams(
            vmem_limit_bytes=63 * 1024 * 1024),
    )(keys)

    bt2 = min(1024, batch)
    hb2 = min(1024, hidden)
    nb2, nh2 = batch // bt2, hidden // hb2
    decoded = pl.pallas_call(
        functools.partial(_dec_kernel, nh=nh2, nb=nb2, bt=bt2),
        grid=(nb2, nh2),
        in_specs=[
            pl.BlockSpec((bt2, hb2), lambda i, j: (i, j)),
            pl.BlockSpec((din, hb2), lambda i, j: (0, j)),
        ],
        out_specs=pl.BlockSpec(memory_space=pltpu.MemorySpace.HBM),
        out_shape=jax.ShapeDtypeStruct((batch, din), jnp.float32),
        scratch_shapes=[pltpu.VMEM((bt2, din), jnp.float32),
                        pltpu.SemaphoreType.DMA],
        compiler_params=pltpu.CompilerParams(
            vmem_limit_bytes=63 * 1024 * 1024),
    )(sparse, W_dec.astype(jnp.bfloat16))

    return (decoded, sparse)


# speedup vs baseline: 1.8466x; 1.8466x over previous
import functools

import jax
import jax.numpy as jnp
from jax import lax
from jax.experimental import pallas as pl
from jax.experimental.pallas import tpu as pltpu

K_TOP = 64
MIN32 = -(2 ** 31)
POS_MASK = 0x7FFFFFFF


def _key_of_bits(b):
    return jnp.where(b >= 0, b, b ^ POS_MASK)


def _enc_kernel(x_ref, w_ref, keys_ref):
    enc = lax.dot_general(
        x_ref[...], w_ref[...], (((1,), (1,)), ((), ())),
        preferred_element_type=jnp.float32)
    keys_ref[...] = _key_of_bits(lax.bitcast_convert_type(enc, jnp.int32))


def _topk_mask_kernel(keys_ref, out_ref, outbf_ref, m_ref, *, bt, hidden):
    ns = 8
    g = hidden // ns

    m = keys_ref[:, pl.ds(0, g)]
    for s in range(1, ns):
        m = jnp.maximum(m, keys_ref[:, pl.ds(s * g, g)])
    m_ref[...] = m
    m1 = jnp.max(m, axis=1, keepdims=True)

    def m_count(t_signed):
        return jnp.sum((m_ref[...] >= t_signed).astype(jnp.int32),
                       axis=1, keepdims=True)

    def full_count(t_signed):
        def chunk_body(c, part):
            kc = keys_ref[:, pl.ds(c * g, g)]
            return part + (kc >= t_signed).astype(jnp.int32)
        part = lax.fori_loop(0, ns, chunk_body, jnp.zeros((bt, g), jnp.int32))
        return jnp.sum(part, axis=1, keepdims=True)

    zero = jnp.zeros((bt, 1), jnp.int32)

    def m_bit_body(state):
        t, pa, fa, ta, pb, fb, tb = state
        bitval = lax.shift_left(jnp.int32(1), 31 - t)
        canda = pa | bitval
        ca = m_count(canda ^ MIN32)
        fa_now = jnp.where((ca == K_TOP) & (fa == 0), 1, 0)
        ta = jnp.where(fa_now == 1, canda ^ MIN32, ta)
        fa = fa | fa_now
        pa = jnp.where(ca >= K_TOP, canda, pa)
        candb = pb | bitval
        cb = m_count(candb ^ MIN32)
        fb_now = jnp.where((cb == ns) & (fb == 0), 1, 0)
        tb = jnp.where(fb_now == 1, candb ^ MIN32, tb)
        fb = fb | fb_now
        pb = jnp.where(cb >= ns + 1, candb, pb)
        return (t + 1, pa, fa, ta, pb, fb, tb)

    def m_bit_cond(state):
        t, _, fa, _, _, fb, _ = state
        return (t < 32) & (jnp.min(fa & fb) == 0)

    _, pa, fa, ta, pb, fb, tb = lax.while_loop(
        m_bit_cond, m_bit_body, (jnp.int32(0), zero, zero, zero,
                                 zero, zero, zero))
    lo = jnp.where(fa == 1, ta, pa ^ MIN32)
    hi = jnp.minimum(jnp.where(fb == 1, tb, (pb + 1) ^ MIN32), m1 + 1)

    tf = zero
    c_lo = full_count(lo)
    found = jnp.where(c_lo == K_TOP, 1, 0)
    tf = jnp.where(found == 1, lo, tf)
    c_hi = full_count(hi)
    hi_hit = jnp.where((c_hi == K_TOP) & (found == 0), 1, 0)
    tf = jnp.where(hi_hit == 1, hi, tf)
    found = found | hi_hit

    def bis_body(state):
        it, lo, hi, found, tf = state
        mid = (lo >> 1) + (hi >> 1) + (lo & hi & 1)
        collapsed = jnp.where((mid == lo) & (found == 0), 1, 0)
        tf = jnp.where(collapsed == 1, lo, tf)
        found = found | collapsed
        c = full_count(mid)
        hit = jnp.where((c == K_TOP) & (found == 0), 1, 0)
        tf = jnp.where(hit == 1, mid, tf)
        upd = (found | hit) == 0
        lo = jnp.where(upd & (c >= K_TOP), mid, lo)
        hi = jnp.where(upd & (c < K_TOP), mid, hi)
        found = found | hit
        return (it + 1, lo, hi, found, tf)

    def bis_cond(state):
        it, lo, hi, found, _ = state
        return (it < 40) & (jnp.min(found) == 0)

    _, lo, _, found, tf = lax.while_loop(
        bis_cond, bis_body, (jnp.int32(0), lo, hi, found, tf))
    t_signed = jnp.where(found == 1, tf, lo)

    def mask_body(c, carry):
        sl = pl.ds(c * g, g)
        kc = keys_ref[:, sl]
        val = lax.bitcast_convert_type(_key_of_bits(kc), jnp.float32)
        sp = jnp.where(kc >= t_signed, val, 0.0)
        out_ref[:, sl] = sp
        outbf_ref[:, sl] = sp.astype(jnp.bfloat16)
        return carry

    lax.fori_loop(0, ns, mask_body, 0)


def _dec_kernel(s_ref, w_ref, out_ref, acc_ref, sem, *, nh, nb, bt):
    i = pl.program_id(0)
    j = pl.program_id(1)

    @pl.when((j == 0) & (i > 0))
    def _wait_prev():
        pltpu.make_async_copy(
            acc_ref, out_ref.at[pl.ds((i - 1) * bt, bt), :], sem).wait()

    prod = lax.dot_general(
        s_ref[...], w_ref[...],
        (((1,), (1,)), ((), ())), preferred_element_type=jnp.float32)

    @pl.when(j == 0)
    def _init():
        acc_ref[...] = prod

    @pl.when(j != 0)
    def _acc():
        acc_ref[...] += prod

    @pl.when(j == nh - 1)
    def _flush():
        copy = pltpu.make_async_copy(
            acc_ref, out_ref.at[pl.ds(i * bt, bt), :], sem)
        copy.start()

        @pl.when(i == nb - 1)
        def _wait_last():
            copy.wait()


@jax.jit
def kernel(x, W_enc, W_dec):
    batch, din = x.shape
    hidden = W_enc.shape[0]

    bt = min(1024, batch)
    hb = min(512, hidden)
    keys = pl.pallas_call(
        _enc_kernel,
        grid=(batch // bt, hidden // hb),
        in_specs=[
            pl.BlockSpec((bt, din), lambda i, j: (i, 0)),
            pl.BlockSpec((hb, din), lambda i, j: (j, 0)),
        ],
        out_specs=pl.BlockSpec((bt, hb), lambda i, j: (i, j)),
        out_shape=jax.ShapeDtypeStruct((batch, hidden), jnp.int32),
        compiler_params=pltpu.CompilerParams(
            vmem_limit_bytes=63 * 1024 * 1024),
    )(x, W_enc)

    bta = min(128, batch)
    sparse, sparse_bf = pl.pallas_call(
        functools.partial(_topk_mask_kernel, bt=bta, hidden=hidden),
        grid=(batch // bta,),
        in_specs=[pl.BlockSpec((bta, hidden), lambda i: (i, 0))],
        out_specs=[pl.BlockSpec((bta, hidden), lambda i: (i, 0)),
                   pl.BlockSpec((bta, hidden), lambda i: (i, 0))],
        out_shape=[jax.ShapeDtypeStruct((batch, hidden), jnp.float32),
                   jax.ShapeDtypeStruct((batch, hidden), jnp.bfloat16)],
        scratch_shapes=[pltpu.VMEM((bta, hidden // 8), jnp.int32)],
        compiler_params=pltpu.CompilerParams(
            vmem_limit_bytes=63 * 1024 * 1024),
    )(keys)

    bt2 = min(1024, batch)
    hb2 = min(1024, hidden)
    nb2, nh2 = batch // bt2, hidden // hb2
    decoded = pl.pallas_call(
        functools.partial(_dec_kernel, nh=nh2, nb=nb2, bt=bt2),
        grid=(nb2, nh2),
        in_specs=[
            pl.BlockSpec((bt2, hb2), lambda i, j: (i, j)),
            pl.BlockSpec((din, hb2), lambda i, j: (0, j)),
        ],
        out_specs=pl.BlockSpec(memory_space=pltpu.MemorySpace.HBM),
        out_shape=jax.ShapeDtypeStruct((batch, din), jnp.float32),
        scratch_shapes=[pltpu.VMEM((bt2, din), jnp.float32),
                        pltpu.SemaphoreType.DMA],
        compiler_params=pltpu.CompilerParams(
            vmem_limit_bytes=63 * 1024 * 1024),
    )(sparse_bf, W_dec.astype(jnp.bfloat16))

    return (decoded, sparse)


# speedup vs baseline: 2.2478x; 1.2173x over previous
import functools

import jax
import jax.numpy as jnp
from jax import lax
from jax.experimental import pallas as pl
from jax.experimental.pallas import tpu as pltpu

K_TOP = 64
MIN32 = -(2 ** 31)
POS_MASK = 0x7FFFFFFF


def _key_of_bits(b):
    return jnp.where(b >= 0, b, b ^ POS_MASK)


def _enc_kernel(x_ref, w_ref, keys_ref):
    enc = lax.dot_general(
        x_ref[...], w_ref[...], (((1,), (1,)), ((), ())),
        preferred_element_type=jnp.float32)
    keys_ref[...] = _key_of_bits(lax.bitcast_convert_type(enc, jnp.int32))


def _topk_mask_kernel(keys_ref, out_ref, outbf_ref, m_ref, *, bt, hidden):
    ns = 8
    g = hidden // ns

    m = keys_ref[:, pl.ds(0, g)]
    for s in range(1, ns):
        m = jnp.maximum(m, keys_ref[:, pl.ds(s * g, g)])
    m_ref[...] = m
    m1 = jnp.max(m, axis=1, keepdims=True)

    def m_count(t_signed):
        return jnp.sum((m_ref[...] >= t_signed).astype(jnp.int32),
                       axis=1, keepdims=True)

    def full_count(t_signed):
        def chunk_body(c, part):
            kc = keys_ref[:, pl.ds(c * g, g)]
            return part + (kc >= t_signed).astype(jnp.int32)
        part = lax.fori_loop(0, ns, chunk_body, jnp.zeros((bt, g), jnp.int32))
        return jnp.sum(part, axis=1, keepdims=True)

    zero = jnp.zeros((bt, 1), jnp.int32)

    def m_bit_body(state):
        t, pa, fa, ta = state
        bitval = lax.shift_left(jnp.int32(1), 31 - t)
        canda = pa | bitval
        ca = m_count(canda ^ MIN32)
        fa_now = jnp.where((ca == K_TOP) & (fa == 0), 1, 0)
        ta = jnp.where(fa_now == 1, canda ^ MIN32, ta)
        fa = fa | fa_now
        pa = jnp.where(ca >= K_TOP, canda, pa)
        return (t + 1, pa, fa, ta)

    def m_bit_cond(state):
        t, _, fa, _ = state
        return (t < 32) & (jnp.min(fa) == 0)

    _, pa, fa, ta = lax.while_loop(
        m_bit_cond, m_bit_body, (jnp.int32(0), zero, zero, zero))
    lo = jnp.where(fa == 1, ta, pa ^ MIN32)
    hi = m1 + 1
    c_hi = zero

    tf = zero
    c_lo = full_count(lo)
    found = jnp.where(c_lo == K_TOP, 1, 0)
    tf = jnp.where(found == 1, lo, tf)

    def bis_body(state):
        it, lo, hi, c_lo, c_hi, found, tf = state
        bis_mid = (lo >> 1) + (hi >> 1) + (lo & hi & 1)
        collapsed = jnp.where((bis_mid == lo) & (found == 0), 1, 0)
        tf = jnp.where(collapsed == 1, lo, tf)
        d = hi - lo
        denom = jnp.maximum(c_lo - c_hi, 1).astype(jnp.float32)
        frac = jnp.clip((c_lo - K_TOP).astype(jnp.float32) / denom,
                        0.03, 0.97)
        step = jnp.maximum((frac * d.astype(jnp.float32)).astype(jnp.int32),
                           1)
        mid_i = jnp.minimum(lo + step, hi - 1)
        use_i = (d > 1) & ((it % 3) != 2)
        mid = jnp.where(use_i, mid_i, bis_mid)
        c = full_count(mid)
        done0 = found | collapsed
        hit = jnp.where((c == K_TOP) & (done0 == 0), 1, 0)
        tf = jnp.where(hit == 1, mid, tf)
        upd = (done0 | hit) == 0
        geq = c >= K_TOP
        lo = jnp.where(upd & geq, mid, lo)
        c_lo = jnp.where(upd & geq, c, c_lo)
        hi = jnp.where(upd & ~geq, mid, hi)
        c_hi = jnp.where(upd & ~geq, c, c_hi)
        found = done0 | hit
        return (it + 1, lo, hi, c_lo, c_hi, found, tf)

    def bis_cond(state):
        it = state[0]
        found = state[5]
        return (it < 48) & (jnp.min(found) == 0)

    _, lo, _, _, _, found, tf = lax.while_loop(
        bis_cond, bis_body, (jnp.int32(0), lo, hi, c_lo, c_hi, found, tf))
    t_signed = jnp.where(found == 1, tf, lo)

    def mask_body(c, carry):
        sl = pl.ds(c * g, g)
        kc = keys_ref[:, sl]
        val = lax.bitcast_convert_type(_key_of_bits(kc), jnp.float32)
        sp = jnp.where(kc >= t_signed, val, 0.0)
        out_ref[:, sl] = sp
        outbf_ref[:, sl] = sp.astype(jnp.bfloat16)
        return carry

    lax.fori_loop(0, ns, mask_body, 0)


def _dec_kernel(s_ref, w_ref, out_ref, acc_ref, sem, *, nh, nb, bt):
    i = pl.program_id(0)
    j = pl.program_id(1)

    @pl.when((j == 0) & (i > 0))
    def _wait_prev():
        pltpu.make_async_copy(
            acc_ref, out_ref.at[pl.ds((i - 1) * bt, bt), :], sem).wait()

    prod = lax.dot_general(
        s_ref[...], w_ref[...],
        (((1,), (1,)), ((), ())), preferred_element_type=jnp.float32)

    @pl.when(j == 0)
    def _init():
        acc_ref[...] = prod

    @pl.when(j != 0)
    def _acc():
        acc_ref[...] += prod

    @pl.when(j == nh - 1)
    def _flush():
        copy = pltpu.make_async_copy(
            acc_ref, out_ref.at[pl.ds(i * bt, bt), :], sem)
        copy.start()

        @pl.when(i == nb - 1)
        def _wait_last():
            copy.wait()


@jax.jit
def kernel(x, W_enc, W_dec):
    batch, din = x.shape
    hidden = W_enc.shape[0]

    bt = min(1024, batch)
    hb = min(512, hidden)
    keys = pl.pallas_call(
        _enc_kernel,
        grid=(batch // bt, hidden // hb),
        in_specs=[
            pl.BlockSpec((bt, din), lambda i, j: (i, 0)),
            pl.BlockSpec((hb, din), lambda i, j: (j, 0)),
        ],
        out_specs=pl.BlockSpec((bt, hb), lambda i, j: (i, j)),
        out_shape=jax.ShapeDtypeStruct((batch, hidden), jnp.int32),
        compiler_params=pltpu.CompilerParams(
            vmem_limit_bytes=63 * 1024 * 1024),
    )(x, W_enc)

    bta = min(128, batch)
    sparse, sparse_bf = pl.pallas_call(
        functools.partial(_topk_mask_kernel, bt=bta, hidden=hidden),
        grid=(batch // bta,),
        in_specs=[pl.BlockSpec((bta, hidden), lambda i: (i, 0))],
        out_specs=[pl.BlockSpec((bta, hidden), lambda i: (i, 0)),
                   pl.BlockSpec((bta, hidden), lambda i: (i, 0))],
        out_shape=[jax.ShapeDtypeStruct((batch, hidden), jnp.float32),
                   jax.ShapeDtypeStruct((batch, hidden), jnp.bfloat16)],
        scratch_shapes=[pltpu.VMEM((bta, hidden // 8), jnp.int32)],
        compiler_params=pltpu.CompilerParams(
            vmem_limit_bytes=63 * 1024 * 1024),
    )(keys)

    bt2 = min(1024, batch)
    hb2 = min(1024, hidden)
    nb2, nh2 = batch // bt2, hidden // hb2
    decoded = pl.pallas_call(
        functools.partial(_dec_kernel, nh=nh2, nb=nb2, bt=bt2),
        grid=(nb2, nh2),
        in_specs=[
            pl.BlockSpec((bt2, hb2), lambda i, j: (i, j)),
            pl.BlockSpec((din, hb2), lambda i, j: (0, j)),
        ],
        out_specs=pl.BlockSpec(memory_space=pltpu.MemorySpace.HBM),
        out_shape=jax.ShapeDtypeStruct((batch, din), jnp.float32),
        scratch_shapes=[pltpu.VMEM((bt2, din), jnp.float32),
                        pltpu.SemaphoreType.DMA],
        compiler_params=pltpu.CompilerParams(
            vmem_limit_bytes=63 * 1024 * 1024),
    )(sparse_bf, W_dec.astype(jnp.bfloat16))

    return (decoded, sparse)


# speedup vs baseline: 2.2783x; 1.0136x over previous
import functools

import jax
import jax.numpy as jnp
from jax import lax
from jax.experimental import pallas as pl
from jax.experimental.pallas import tpu as pltpu

K_TOP = 64
MIN32 = -(2 ** 31)
POS_MASK = 0x7FFFFFFF


def _key_of_bits(b):
    return jnp.where(b >= 0, b, b ^ POS_MASK)


def _enc_kernel(x_ref, w_ref, keys_ref):
    enc = lax.dot_general(
        x_ref[...], w_ref[...], (((1,), (1,)), ((), ())),
        preferred_element_type=jnp.float32)
    keys_ref[...] = _key_of_bits(lax.bitcast_convert_type(enc, jnp.int32))


def _topk_mask_kernel(keys_ref, out_ref, outbf_ref, m_ref, *, bt, hidden):
    ns = 8
    g = hidden // ns

    m = keys_ref[:, pl.ds(0, g)]
    for s in range(1, ns):
        m = jnp.maximum(m, keys_ref[:, pl.ds(s * g, g)])
    m_ref[...] = m
    m1 = jnp.max(m, axis=1, keepdims=True)

    def m_count(t_signed):
        return jnp.sum((m_ref[...] >= t_signed).astype(jnp.int32),
                       axis=1, keepdims=True)

    def full_count(t_signed):
        def chunk_body(c, part):
            kc = keys_ref[:, pl.ds(c * g, g)]
            return part + (kc >= t_signed).astype(jnp.int32)
        part = lax.fori_loop(0, ns, chunk_body, jnp.zeros((bt, g), jnp.int32))
        return jnp.sum(part, axis=1, keepdims=True)

    def _mid(a, b):
        return (a >> 1) + (b >> 1) + (a & b & 1)

    zero = jnp.zeros((bt, 1), jnp.int32)

    def m_bit_body(state):
        t, pa, fa, ta = state
        bitval = lax.shift_left(jnp.int32(1), 31 - t)
        canda = pa | bitval
        ca = m_count(canda ^ MIN32)
        fa_now = jnp.where((ca == K_TOP) & (fa == 0), 1, 0)
        ta = jnp.where(fa_now == 1, canda ^ MIN32, ta)
        fa = fa | fa_now
        pa = jnp.where(ca >= K_TOP, canda, pa)
        return (t + 1, pa, fa, ta)

    def m_bit_cond(state):
        t, _, fa, _ = state
        return (t < 32) & (jnp.min(fa) == 0)

    _, pa, fa, ta = lax.while_loop(
        m_bit_cond, m_bit_body, (jnp.int32(0), zero, zero, zero))
    lo = jnp.where(fa == 1, ta, pa ^ MIN32)
    hi = m1 + 1
    c_hi = zero

    tf = zero
    c_lo = full_count(lo)
    found = jnp.where(c_lo == K_TOP, 1, 0)
    tf = jnp.where(found == 1, lo, tf)

    def bis_body(state):
        it, lo, hi, c_lo, c_hi, found, tf = state
        bis_mid = _mid(lo, hi)
        collapsed = jnp.where((bis_mid == lo) & (found == 0), 1, 0)
        tf = jnp.where(collapsed == 1, lo, tf)
        d = hi - lo
        denom = jnp.maximum(c_lo - c_hi, 1).astype(jnp.float32)
        frac = jnp.clip((c_lo - K_TOP).astype(jnp.float32) / denom,
                        0.03, 0.97)
        step = jnp.maximum((frac * d.astype(jnp.float32)).astype(jnp.int32),
                           1)
        mid_i = jnp.minimum(lo + step, hi - 1)
        use_i = (d > 1) & ((it % 3) != 2)
        mid = jnp.where(use_i, mid_i, bis_mid)
        c = full_count(mid)
        done0 = found | collapsed
        hit = jnp.where((c == K_TOP) & (done0 == 0), 1, 0)
        tf = jnp.where(hit == 1, mid, tf)
        upd = (done0 | hit) == 0
        geq = c >= K_TOP
        lo = jnp.where(upd & geq, mid, lo)
        c_lo = jnp.where(upd & geq, c, c_lo)
        hi = jnp.where(upd & ~geq, mid, hi)
        c_hi = jnp.where(upd & ~geq, c, c_hi)
        found = done0 | hit
        return (it + 1, lo, hi, c_lo, c_hi, found, tf)

    def bis_cond(state):
        it = state[0]
        found = state[5]
        return (it < 48) & (jnp.min(found) == 0)

    _, lo, _, _, _, found, tf = lax.while_loop(
        bis_cond, bis_body, (jnp.int32(0), lo, hi, c_lo, c_hi, found, tf))
    t_signed = jnp.where(found == 1, tf, lo)

    def mask_body(c, carry):
        sl = pl.ds(c * g, g)
        kc = keys_ref[:, sl]
        val = lax.bitcast_convert_type(_key_of_bits(kc), jnp.float32)
        sp = jnp.where(kc >= t_signed, val, 0.0)
        out_ref[:, sl] = sp
        outbf_ref[:, sl] = sp.astype(jnp.bfloat16)
        return carry

    lax.fori_loop(0, ns, mask_body, 0)


def _dec_kernel(s_ref, w_ref, out_ref, acc_ref, sem, *, nh, nb, bt, din):
    i = pl.program_id(0)
    j = pl.program_id(1)

    @pl.when((j == 0) & (i > 0))
    def _wait_prev():
        pltpu.make_async_copy(
            acc_ref, out_ref.at[pl.ds((i - 1) * bt, bt), :], sem).wait()

    nsl = 4
    sub = din // nsl
    for n in range(nsl):
        prod = lax.dot_general(
            s_ref[...], w_ref[pl.ds(n * sub, sub), :],
            (((1,), (1,)), ((), ())), preferred_element_type=jnp.float32)

        @pl.when(j == 0)
        def _init(prod=prod, n=n):
            acc_ref[:, pl.ds(n * sub, sub)] = prod

        @pl.when(j != 0)
        def _acc(prod=prod, n=n):
            acc_ref[:, pl.ds(n * sub, sub)] += prod

    @pl.when(j == nh - 1)
    def _flush():
        copy = pltpu.make_async_copy(
            acc_ref, out_ref.at[pl.ds(i * bt, bt), :], sem)
        copy.start()

        @pl.when(i == nb - 1)
        def _wait_last():
            copy.wait()


@jax.jit
def kernel(x, W_enc, W_dec):
    batch, din = x.shape
    hidden = W_enc.shape[0]

    bt = min(1024, batch)
    hb = min(512, hidden)
    keys = pl.pallas_call(
        _enc_kernel,
        grid=(batch // bt, hidden // hb),
        in_specs=[
            pl.BlockSpec((bt, din), lambda i, j: (i, 0)),
            pl.BlockSpec((hb, din), lambda i, j: (j, 0)),
        ],
        out_specs=pl.BlockSpec((bt, hb), lambda i, j: (i, j)),
        out_shape=jax.ShapeDtypeStruct((batch, hidden), jnp.int32),
        compiler_params=pltpu.CompilerParams(
            vmem_limit_bytes=63 * 1024 * 1024),
    )(x, W_enc)

    bta = min(128, batch)
    sparse, sparse_bf = pl.pallas_call(
        functools.partial(_topk_mask_kernel, bt=bta, hidden=hidden),
        grid=(batch // bta,),
        in_specs=[pl.BlockSpec((bta, hidden), lambda i: (i, 0))],
        out_specs=[pl.BlockSpec((bta, hidden), lambda i: (i, 0)),
                   pl.BlockSpec((bta, hidden), lambda i: (i, 0))],
        out_shape=[jax.ShapeDtypeStruct((batch, hidden), jnp.float32),
                   jax.ShapeDtypeStruct((batch, hidden), jnp.bfloat16)],
        scratch_shapes=[pltpu.VMEM((bta, hidden // 8), jnp.int32)],
        compiler_params=pltpu.CompilerParams(
            vmem_limit_bytes=63 * 1024 * 1024),
    )(keys)

    bt2 = min(1024, batch)
    hb2 = min(2048, hidden)
    nb2, nh2 = batch // bt2, hidden // hb2
    decoded = pl.pallas_call(
        functools.partial(_dec_kernel, nh=nh2, nb=nb2, bt=bt2, din=din),
        grid=(nb2, nh2),
        in_specs=[
            pl.BlockSpec((bt2, hb2), lambda i, j: (i, j)),
            pl.BlockSpec((din, hb2), lambda i, j: (0, j)),
        ],
        out_specs=pl.BlockSpec(memory_space=pltpu.MemorySpace.HBM),
        out_shape=jax.ShapeDtypeStruct((batch, din), jnp.float32),
        scratch_shapes=[pltpu.VMEM((bt2, din), jnp.float32),
                        pltpu.SemaphoreType.DMA],
        compiler_params=pltpu.CompilerParams(
            vmem_limit_bytes=63 * 1024 * 1024),
    )(sparse_bf, W_dec.astype(jnp.bfloat16))

    return (decoded, sparse)
